# baseline (speedup 1.0000x reference)
import functools

import jax
import jax.numpy as jnp
from jax import lax
from jax.experimental import pallas as pl
from jax.experimental.pallas import tpu as pltpu
from jax.experimental.pallas import tpu_sc as plsc

_NUM_CORES = 2
_NUM_SUBCORES = 16
_LANES = 16
_NW = _NUM_CORES * _NUM_SUBCORES

_V = 32
_NDIM = 10
_W_ROW = _V
_PAR_ROW = _V + _NDIM
_PROWS = _PAR_ROW + 3
_EPS = 1e-12
_BETA = 3.0

_MAGIC = 0x5F3759DF


def _sqrt16(x):
    i = lax.bitcast_convert_type(x, jnp.int32)
    i = _MAGIC - lax.shift_right_logical(i, 1)
    y = lax.bitcast_convert_type(i, jnp.float32)
    half_x = 0.5 * x
    for _ in range(3):
        y = y * (1.5 - half_x * y * y)
    return x * y


def _make_sc_kernel(batch):
    bw = batch // _NW
    n_chunks = bw // _LANES

    mesh = plsc.VectorSubcoreMesh(core_axis_name="c", subcore_axis_name="s")

    @functools.partial(
        pl.kernel,
        out_type=jax.ShapeDtypeStruct((batch,), jnp.float32),
        mesh=mesh,
        compiler_params=pltpu.CompilerParams(needs_layout_passes=False),
        scratch_types=[
            pltpu.VMEM((bw, 2), jnp.int32),
            pltpu.VMEM((_PROWS, _LANES), jnp.float32),
            pltpu.VMEM((bw,), jnp.float32),
            pltpu.SemaphoreType.DMA,
            pltpu.SemaphoreType.DMA,
        ],
    )
    def k(idx_hbm, par_hbm, out_hbm, idx_v, par_v, out_v, sem_idx, sem_par):
        c = lax.axis_index("c")
        s = lax.axis_index("s")
        wid = s * _NUM_CORES + c
        base = wid * bw

        idx_cp = pltpu.async_copy(idx_hbm.at[pl.ds(base, bw)], idx_v, sem_idx)
        pltpu.async_copy(par_hbm, par_v, sem_par).wait()

        iota = lax.iota(jnp.int32, _LANES)
        zero_i = jnp.zeros((_LANES,), jnp.int32)
        one_i = jnp.full((_LANES,), 1, jnp.int32)
        dcols = [jnp.full((_LANES,), d, jnp.int32) for d in range(_NDIM)]
        wv = [par_v[_W_ROW + d] for d in range(_NDIM)]
        up = par_v[_PAR_ROW]
        mid = par_v[_PAR_ROW + 1]
        rt = par_v[_PAR_ROW + 2]

        idx_cp.wait()
        for blk in range(n_chunks):
            lid = iota + blk * _LANES
            i0 = plsc.load_gather(idx_v, [lid, zero_i])
            i1 = plsc.load_gather(idx_v, [lid, one_i])
            acc = jnp.zeros((_LANES,), jnp.float32)
            for d in range(_NDIM):
                a = plsc.load_gather(par_v, [i0, dcols[d]])
                b = plsc.load_gather(par_v, [i1, dcols[d]])
                diff = a - b
                acc = acc + wv[d] * (diff * diff)
            dist = _sqrt16(jnp.maximum(acc, _EPS))
            sim = jnp.exp(-_BETA * dist)
            yv = up / (1.0 + jnp.exp(-rt * (sim - mid)))
            out_v[pl.ds(blk * _LANES, _LANES)] = yv

        pltpu.sync_copy(out_v, out_hbm.at[pl.ds(base, bw)])

    return k


def kernel(rate2_stimulus_set, percept, w, upper, midpoint, rate):
    batch = rate2_stimulus_set.shape[0]
    idx = rate2_stimulus_set.astype(jnp.int32)
    par = jnp.zeros((_PROWS, _LANES), jnp.float32)
    par = par.at[: percept.shape[0], :_NDIM].set(percept.astype(jnp.float32))
    par = par.at[_W_ROW : _W_ROW + _NDIM, :].set(
        jnp.broadcast_to(w.astype(jnp.float32)[:, None], (_NDIM, _LANES))
    )
    par = par.at[_PAR_ROW, :].set(jnp.float32(upper))
    par = par.at[_PAR_ROW + 1, :].set(jnp.float32(midpoint))
    par = par.at[_PAR_ROW + 2, :].set(jnp.float32(rate))
    y = _make_sc_kernel(batch)(idx, par)
    return y[:, None]


# speedup vs baseline: 1.1894x; 1.1894x over previous
import functools

import jax
import jax.numpy as jnp
from jax import lax
from jax.experimental import pallas as pl
from jax.experimental.pallas import tpu as pltpu
from jax.experimental.pallas import tpu_sc as plsc

_NUM_CORES = 2
_NUM_SUBCORES = 16
_LANES = 16
_NW = _NUM_CORES * _NUM_SUBCORES

_V = 32
_NDIM = 10
_EPS = 1e-12
_BETA = 3.0

_MAGIC = 0x5F3759DF


def _sqrt16(x):
    i = lax.bitcast_convert_type(x, jnp.int32)
    i = _MAGIC - lax.shift_right_logical(i, 1)
    y = lax.bitcast_convert_type(i, jnp.float32)
    half_x = 0.5 * x
    for _ in range(3):
        y = y * (1.5 - half_x * y * y)
    return x * y


def _make_sc_kernel(batch, n_stim):
    bw = batch // _NW
    n_chunks = bw // _LANES
    tab_per_sub = (_V * _V) // _NUM_SUBCORES

    mesh = plsc.VectorSubcoreMesh(core_axis_name="c", subcore_axis_name="s")

    @functools.partial(
        pl.kernel,
        out_type=jax.ShapeDtypeStruct((batch,), jnp.float32),
        mesh=mesh,
        compiler_params=pltpu.CompilerParams(needs_layout_passes=False),
        scratch_types=[
            pltpu.VMEM((bw, 2), jnp.int32),
            pltpu.VMEM((n_stim, _NDIM), jnp.float32),
            pltpu.VMEM((_NDIM + 3, _LANES), jnp.float32),
            pltpu.VMEM((tab_per_sub,), jnp.float32),
            pltpu.VMEM((_V * _V,), jnp.float32),
            pltpu.VMEM((2, bw), jnp.int32),
            pltpu.VMEM((bw,), jnp.float32),
            pltpu.VMEM_SHARED((_V * _V,), jnp.float32),
            pltpu.SemaphoreType.DMA,
            pltpu.SemaphoreType.DMA,
        ],
    )
    def k(idx_hbm, pc_hbm, par_hbm, out_hbm,
          idx_v, pc_v, par_v, stage_v, ptab_v, ii_v, out_v,
          shared, sem_idx, sem_par):
        c = lax.axis_index("c")
        s = lax.axis_index("s")
        wid = s * _NUM_CORES + c
        base = wid * bw

        idx_cp = pltpu.async_copy(idx_hbm.at[pl.ds(base, bw)], idx_v, sem_idx)
        pc_cp = pltpu.async_copy(pc_hbm, pc_v, sem_par)
        par_cp = pltpu.async_copy(par_hbm, par_v, sem_par)

        iota = lax.iota(jnp.int32, _LANES)
        zero_i = jnp.zeros((_LANES,), jnp.int32)
        one_i = jnp.full((_LANES,), 1, jnp.int32)
        zero_f = jnp.zeros((_LANES,), jnp.float32)

        idx_cp.wait()
        for blk in range(n_chunks):
            lid = iota + blk * _LANES
            ii_v[0, pl.ds(blk * _LANES, _LANES)] = plsc.load_gather(
                idx_v, [lid, zero_i]
            )
            ii_v[1, pl.ds(blk * _LANES, _LANES)] = plsc.load_gather(
                idx_v, [lid, one_i]
            )

        pc_cp.wait()
        par_cp.wait()

        wv = [par_v[d] for d in range(_NDIM)]
        up = par_v[_NDIM]
        mid = par_v[_NDIM + 1]
        rt = par_v[_NDIM + 2]
        max_row = jnp.full((_LANES,), n_stim - 1, jnp.int32)

        tab_base = s * tab_per_sub
        for j in range(tab_per_sub // _LANES):
            flat = iota + (tab_base + j * _LANES)
            r0 = jnp.minimum(lax.shift_right_logical(flat, 5), max_row)
            r1 = jnp.minimum(lax.bitwise_and(flat, _V - 1), max_row)
            acc = zero_f
            for d in range(_NDIM):
                dcol = jnp.full((_LANES,), d, jnp.int32)
                a = plsc.load_gather(pc_v, [r0, dcol])
                b = plsc.load_gather(pc_v, [r1, dcol])
                diff = a - b
                acc = acc + wv[d] * (diff * diff)
            dist = _sqrt16(jnp.maximum(acc, _EPS))
            sim = jnp.exp(-_BETA * dist)
            yv = up / (1.0 + jnp.exp(-rt * (sim - mid)))
            stage_v[pl.ds(j * _LANES, _LANES)] = yv

        pltpu.sync_copy(stage_v, shared.at[pl.ds(tab_base, tab_per_sub)])
        plsc.subcore_barrier()
        pltpu.sync_copy(shared, ptab_v)

        for blk in range(n_chunks):
            off = blk * _LANES
            i0 = ii_v[0, pl.ds(off, _LANES)]
            i1 = ii_v[1, pl.ds(off, _LANES)]
            flat = lax.shift_left(i0, 5) + i1
            out_v[pl.ds(off, _LANES)] = plsc.load_gather(ptab_v, [flat])
        pltpu.sync_copy(out_v, out_hbm.at[pl.ds(base, bw)])

    return k


def kernel(rate2_stimulus_set, percept, w, upper, midpoint, rate):
    batch = rate2_stimulus_set.shape[0]
    idx = rate2_stimulus_set.astype(jnp.int32)
    par = jnp.concatenate(
        [
            jnp.broadcast_to(w.astype(jnp.float32)[:, None], (_NDIM, _LANES)),
            jnp.broadcast_to(
                jnp.stack([upper, midpoint, rate]).astype(jnp.float32)[:, None],
                (3, _LANES),
            ),
        ]
    )
    y = _make_sc_kernel(batch, percept.shape[0])(
        idx, percept.astype(jnp.float32), par
    )
    return y[:, None]


# speedup vs baseline: 1.2595x; 1.0589x over previous
import functools

import jax
import jax.numpy as jnp
from jax import lax
from jax.experimental import pallas as pl
from jax.experimental.pallas import tpu as pltpu
from jax.experimental.pallas import tpu_sc as plsc

_NUM_CORES = 2
_NUM_SUBCORES = 16
_LANES = 16
_NW = _NUM_CORES * _NUM_SUBCORES

_V = 32
_NDIM = 10
_W_ROW = _V
_PAR_ROW = _V + _NDIM
_PROWS = _PAR_ROW + 3
_EPS = 1e-12
_BETA = 3.0

_MAGIC = 0x5F3759DF


def _sqrt16(x):
    i = lax.bitcast_convert_type(x, jnp.int32)
    i = _MAGIC - lax.shift_right_logical(i, 1)
    y = lax.bitcast_convert_type(i, jnp.float32)
    half_x = 0.5 * x
    for _ in range(3):
        y = y * (1.5 - half_x * y * y)
    return x * y


def _make_sc_kernel(batch):
    bw = batch // _NW
    n_chunks = bw // _LANES
    tab_per_sub = (_V * _V) // _NUM_SUBCORES

    mesh = plsc.VectorSubcoreMesh(core_axis_name="c", subcore_axis_name="s")

    @functools.partial(
        pl.kernel,
        out_type=jax.ShapeDtypeStruct((batch,), jnp.float32),
        mesh=mesh,
        compiler_params=pltpu.CompilerParams(needs_layout_passes=False),
        scratch_types=[
            pltpu.VMEM((bw, 2), jnp.int32),
            pltpu.VMEM((_PROWS, _LANES), jnp.float32),
            pltpu.VMEM((tab_per_sub,), jnp.float32),
            pltpu.VMEM((_V * _V,), jnp.float32),
            pltpu.VMEM((bw,), jnp.float32),
            pltpu.VMEM_SHARED((_V * _V,), jnp.float32),
            pltpu.SemaphoreType.DMA,
            pltpu.SemaphoreType.DMA,
        ],
    )
    def k(idx_hbm, par_hbm, out_hbm,
          idx_v, par_v, stage_v, ptab_v, out_v, shared, sem_idx, sem_par):
        c = lax.axis_index("c")
        s = lax.axis_index("s")
        wid = s * _NUM_CORES + c
        base = wid * bw

        idx_cp = pltpu.async_copy(idx_hbm.at[pl.ds(base, bw)], idx_v, sem_idx)
        pltpu.async_copy(par_hbm, par_v, sem_par).wait()

        iota = lax.iota(jnp.int32, _LANES)
        zero_f = jnp.zeros((_LANES,), jnp.float32)

        tab_base = s * tab_per_sub
        wv = [par_v[_W_ROW + d] for d in range(_NDIM)]
        up = par_v[_PAR_ROW]
        mid = par_v[_PAR_ROW + 1]
        rt = par_v[_PAR_ROW + 2]
        for j in range(tab_per_sub // _LANES):
            flat = iota + (tab_base + j * _LANES)
            r0 = lax.shift_right_logical(flat, 5)
            r1 = lax.bitwise_and(flat, _V - 1)
            acc = zero_f
            for d in range(_NDIM):
                dcol = jnp.full((_LANES,), d, jnp.int32)
                a = plsc.load_gather(par_v, [r0, dcol])
                b = plsc.load_gather(par_v, [r1, dcol])
                diff = a - b
                acc = acc + wv[d] * (diff * diff)
            dist = _sqrt16(jnp.maximum(acc, _EPS))
            sim = jnp.exp(-_BETA * dist)
            yv = up / (1.0 + jnp.exp(-rt * (sim - mid)))
            stage_v[pl.ds(j * _LANES, _LANES)] = yv

        pltpu.sync_copy(stage_v, shared.at[pl.ds(tab_base, tab_per_sub)])
        plsc.subcore_barrier()
        pltpu.sync_copy(shared, ptab_v)

        idx_cp.wait()
        zero_i = jnp.zeros((_LANES,), jnp.int32)
        one_i = jnp.full((_LANES,), 1, jnp.int32)
        for blk in range(n_chunks):
            lid = iota + blk * _LANES
            i0 = plsc.load_gather(idx_v, [lid, zero_i])
            i1 = plsc.load_gather(idx_v, [lid, one_i])
            flat = lax.shift_left(i0, 5) + i1
            out_v[pl.ds(blk * _LANES, _LANES)] = plsc.load_gather(ptab_v, [flat])

        pltpu.sync_copy(out_v, out_hbm.at[pl.ds(base, bw)])

    return k


def kernel(rate2_stimulus_set, percept, w, upper, midpoint, rate):
    batch = rate2_stimulus_set.shape[0]
    idx = rate2_stimulus_set.astype(jnp.int32)
    par = jnp.zeros((_PROWS, _LANES), jnp.float32)
    par = par.at[: percept.shape[0], :_NDIM].set(percept.astype(jnp.float32))
    par = par.at[_W_ROW : _W_ROW + _NDIM, :].set(
        jnp.broadcast_to(w.astype(jnp.float32)[:, None], (_NDIM, _LANES))
    )
    par = par.at[_PAR_ROW, :].set(jnp.float32(upper))
    par = par.at[_PAR_ROW + 1, :].set(jnp.float32(midpoint))
    par = par.at[_PAR_ROW + 2, :].set(jnp.float32(rate))
    y = _make_sc_kernel(batch)(idx, par)
    return y[:, None]
